# baseline (speedup 1.0000x reference)
import jax
import jax.numpy as jnp
from jax.experimental import pallas as pl

N = 100000


def _gc(x, edge_index, W_rel, W_root, b):
    src = edge_index[0]
    dst = edge_index[1]
    msg = jnp.take(x, src, axis=0)
    agg = jax.ops.segment_sum(msg, dst, num_segments=N)
    return agg @ W_rel.T + b + x @ W_root.T


def kernel(x, edge_index, W_rel1, W_root1, b1, W_rel2, W_root2, b2, W_rel3, W_root3, b3, Wm1, bm1, Wm2, bm2):
    h = jax.nn.relu(_gc(x, edge_index, W_rel1, W_root1, b1))
    h = jax.nn.relu(_gc(h, edge_index, W_rel2, W_root2, b2))
    h = jax.nn.relu(_gc(h, edge_index, W_rel3, W_root3, b3))
    q = jax.nn.relu(h @ Wm1.T + bm1) @ Wm2.T + bm2
    return (h, q)


# speedup vs baseline: 25.2382x; 25.2382x over previous
import functools

import jax
import jax.numpy as jnp
from jax import lax
from jax.experimental import pallas as pl
from jax.experimental.pallas import tpu as pltpu
from jax.experimental.pallas import tpu_sc as plsc

N = 100000
E = 1600000
F = 16

NC = 2
NS = 16
NW = NC * NS

CH = 128
KCH = 8
NCHUNK = E // CH
BASE_C = NCHUNK // NW
EXTRA = NCHUNK - BASE_C * NW
FULLB = BASE_C // KCH
N_PAD = 100096
RPS = N_PAD // NS

PACK = 128 // F
NPK = N_PAD * F // 128
BRP = 736


def _sc_mesh():
    return plsc.VectorSubcoreMesh(core_axis_name="c", subcore_axis_name="s")


@functools.partial(
    pl.kernel,
    out_type=jax.ShapeDtypeStruct((NC, N_PAD, F), jnp.float32),
    mesh=_sc_mesh(),
    compiler_params=pltpu.CompilerParams(use_tc_tiling_on_sc=False),
    scratch_types=[
        pltpu.VMEM((2, KCH, CH), jnp.int32),
        pltpu.VMEM((KCH, CH, F), jnp.float32),
        pltpu.VMEM_SHARED((N_PAD, F), jnp.float32),
        pltpu.SemaphoreType.DMA,
    ],
)
def _sc_segment_sum(y_hbm, ei_hbm, zero_hbm, out_hbm, idx_v, msg_v, agg_sh, sem):
    cid = lax.axis_index("c")
    sid = lax.axis_index("s")
    wid = cid * NS + sid

    pltpu.sync_copy(zero_hbm, agg_sh.at[pl.ds(sid * RPS, RPS)])
    plsc.subcore_barrier()

    nc = jnp.where(wid < EXTRA, BASE_C + 1, BASE_C)
    c0 = wid * BASE_C + jnp.minimum(wid, EXTRA)

    @pl.loop(0, FULLB)
    def _(i):
        r = c0 + i * KCH
        pltpu.sync_copy(ei_hbm.at[0].at[pl.ds(r, KCH)], idx_v.at[0])
        pltpu.sync_copy(ei_hbm.at[1].at[pl.ds(r, KCH)], idx_v.at[1])
        copies = [
            pltpu.async_copy(y_hbm.at[idx_v.at[0].at[j]], msg_v.at[j], sem)
            for j in range(KCH)
        ]
        for c in copies:
            c.wait()
        for j in range(KCH):
            pltpu.sync_copy(msg_v.at[j], agg_sh.at[idx_v.at[1].at[j]], add=True)

    for j in range(KCH - 1):
        t = FULLB * KCH + j

        @pl.when(t < nc)
        def _():
            r = c0 + t
            pltpu.sync_copy(ei_hbm.at[0].at[pl.ds(r, 1)],
                            idx_v.at[0].at[pl.ds(j, 1)])
            pltpu.sync_copy(ei_hbm.at[1].at[pl.ds(r, 1)],
                            idx_v.at[1].at[pl.ds(j, 1)])
            pltpu.async_copy(y_hbm.at[idx_v.at[0].at[j]], msg_v.at[j],
                             sem).wait()
            pltpu.sync_copy(msg_v.at[j], agg_sh.at[idx_v.at[1].at[j]],
                            add=True)

    plsc.subcore_barrier()
    pltpu.sync_copy(agg_sh.at[pl.ds(sid * RPS, RPS)],
                    out_hbm.at[cid].at[pl.ds(sid * RPS, RPS)])


def _bdot(a, bT):
    return jnp.dot(a.astype(jnp.bfloat16), bT.astype(jnp.bfloat16),
                   preferred_element_type=jnp.float32)


def _tc_layer(parts, h_prev, bd_rel, bd_root, b_tile):
    def body(p_ref, h_ref, wrel_ref, wroot_ref, b_ref, ho_ref):
        agg = p_ref[0] + p_ref[1]
        t = (_bdot(agg, wrel_ref[...]) + b_ref[...]
             + _bdot(h_ref[...], wroot_ref[...]))
        ho_ref[...] = jnp.maximum(t, 0.0)

    return pl.pallas_call(
        body,
        grid=(NPK // BRP,),
        in_specs=[
            pl.BlockSpec((NC, BRP, 128), lambda i: (0, i, 0)),
            pl.BlockSpec((BRP, 128), lambda i: (i, 0)),
            pl.BlockSpec((128, 128), lambda i: (0, 0)),
            pl.BlockSpec((128, 128), lambda i: (0, 0)),
            pl.BlockSpec((1, 128), lambda i: (0, 0)),
        ],
        out_specs=pl.BlockSpec((BRP, 128), lambda i: (i, 0)),
        out_shape=jax.ShapeDtypeStruct((NPK, 128), jnp.float32),
    )(parts, h_prev, bd_rel, bd_root, b_tile)


def _tc_final(parts, h_prev, bd_rel, bd_root, b_tile, bd_m1, bm1_tile,
              wm2_tile, grp, bm2_tile):
    def body(p_ref, h_ref, wrel_ref, wroot_ref, b_ref, wm1_ref, bm1_ref,
             wm2_ref, g_ref, bm2_ref, ho_ref, qo_ref):
        agg = p_ref[0] + p_ref[1]
        t = (_bdot(agg, wrel_ref[...]) + b_ref[...]
             + _bdot(h_ref[...], wroot_ref[...]))
        h = jnp.maximum(t, 0.0)
        ho_ref[...] = h
        m = jnp.maximum(_bdot(h, wm1_ref[...]) + bm1_ref[...], 0.0)
        mb = m.astype(jnp.bfloat16).astype(jnp.float32)
        wb = wm2_ref[...].astype(jnp.bfloat16).astype(jnp.float32)
        prod = mb * wb
        hi = prod.astype(jnp.bfloat16)
        lo = (prod - hi.astype(jnp.float32)).astype(jnp.bfloat16)
        g = g_ref[...].astype(jnp.bfloat16)
        q8 = (jnp.dot(hi, g, preferred_element_type=jnp.float32)
              + jnp.dot(lo, g, preferred_element_type=jnp.float32))
        qo_ref[...] = q8 + bm2_ref[...]

    return pl.pallas_call(
        body,
        grid=(NPK // BRP,),
        in_specs=[
            pl.BlockSpec((NC, BRP, 128), lambda i: (0, i, 0)),
            pl.BlockSpec((BRP, 128), lambda i: (i, 0)),
            pl.BlockSpec((128, 128), lambda i: (0, 0)),
            pl.BlockSpec((128, 128), lambda i: (0, 0)),
            pl.BlockSpec((1, 128), lambda i: (0, 0)),
            pl.BlockSpec((128, 128), lambda i: (0, 0)),
            pl.BlockSpec((1, 128), lambda i: (0, 0)),
            pl.BlockSpec((1, 128), lambda i: (0, 0)),
            pl.BlockSpec((128, PACK), lambda i: (0, 0)),
            pl.BlockSpec((1, PACK), lambda i: (0, 0)),
        ],
        out_specs=[
            pl.BlockSpec((BRP, 128), lambda i: (i, 0)),
            pl.BlockSpec((BRP, PACK), lambda i: (i, 0)),
        ],
        out_shape=[
            jax.ShapeDtypeStruct((NPK, 128), jnp.float32),
            jax.ShapeDtypeStruct((NPK, PACK), jnp.float32),
        ],
    )(parts, h_prev, bd_rel, bd_root, b_tile, bd_m1, bm1_tile, wm2_tile,
      grp, bm2_tile)


def _bd(W):
    WT = W.T
    if WT.shape[0] < F:
        WT = jnp.pad(WT, ((0, F - WT.shape[0]), (0, 0)))
    return jnp.kron(jnp.eye(PACK, dtype=jnp.float32), WT)


def kernel(x, edge_index, W_rel1, W_root1, b1, W_rel2, W_root2, b2,
           W_rel3, W_root3, b3, Wm1, bm1, Wm2, bm2):
    ei = edge_index.reshape(2, NCHUNK, CH)
    zeros_sub = jnp.zeros((RPS, F), jnp.float32)

    def seg(h_packed):
        p = _sc_segment_sum(h_packed.reshape(N_PAD, F), ei, zeros_sub)
        return p.reshape(NC, NPK, 128)

    x16 = jnp.pad(x, ((0, N_PAD - N), (0, F - x.shape[1]))).reshape(NPK, 128)

    def tile(v):
        return jnp.tile(v.reshape(-1), PACK).reshape(1, 128)

    grp = jnp.kron(jnp.eye(PACK, dtype=jnp.float32),
                   jnp.ones((F, 1), jnp.float32))
    bm2_tile = jnp.tile(bm2, PACK).reshape(1, PACK)

    p0 = seg(x16)
    h1 = _tc_layer(p0, x16, _bd(W_rel1), _bd(W_root1), tile(b1))
    p1 = seg(h1)
    h2 = _tc_layer(p1, h1, _bd(W_rel2), _bd(W_root2), tile(b2))
    p2 = seg(h2)
    h3p, qp = _tc_final(p2, h2, _bd(W_rel3), _bd(W_root3), tile(b3),
                        _bd(Wm1), tile(bm1), tile(Wm2), grp, bm2_tile)
    h3 = h3p.reshape(N_PAD, F)[:N]
    q = qp.reshape(N_PAD, 1)[:N]
    return (h3, q)


# speedup vs baseline: 36.0813x; 1.4296x over previous
import functools

import jax
import jax.numpy as jnp
from jax import lax
from jax.experimental import pallas as pl
from jax.experimental.pallas import tpu as pltpu
from jax.experimental.pallas import tpu_sc as plsc

N = 100000
E = 1600000
F = 16

NC = 2
NS = 16
NW = NC * NS

CH = 128
KCH = 5
NCHUNK = E // CH
BASE_C = NCHUNK // NW
EXTRA = NCHUNK - BASE_C * NW
FULLB = BASE_C // KCH
N_PAD = 100096
RPS = N_PAD // NS

PACK = 128 // F
NPK = N_PAD * F // 128
BRP = 736


def _sc_mesh():
    return plsc.VectorSubcoreMesh(core_axis_name="c", subcore_axis_name="s")


@functools.partial(
    pl.kernel,
    out_type=jax.ShapeDtypeStruct((NC, N_PAD, F), jnp.float32),
    mesh=_sc_mesh(),
    compiler_params=pltpu.CompilerParams(use_tc_tiling_on_sc=False),
    scratch_types=[
        pltpu.VMEM((2, 2, KCH, CH), jnp.int32),
        pltpu.VMEM((2, KCH, CH, F), jnp.float32),
        pltpu.VMEM_SHARED((N_PAD, F), jnp.float32),
        pltpu.SemaphoreType.DMA,
        pltpu.SemaphoreType.DMA,
    ],
)
def _sc_segment_sum(y_hbm, ei_hbm, zero_hbm, out_hbm, idx_v, msg_v, agg_sh,
                    semI, semG):
    cid = lax.axis_index("c")
    sid = lax.axis_index("s")
    wid = cid * NS + sid

    pltpu.sync_copy(zero_hbm, agg_sh.at[pl.ds(sid * RPS, RPS)])
    plsc.subcore_barrier()

    nc = jnp.where(wid < EXTRA, BASE_C + 1, BASE_C)
    c0 = wid * BASE_C + jnp.minimum(wid, EXTRA)

    def start_idx(b, buf):
        r = c0 + b * KCH
        pltpu.async_copy(ei_hbm.at[0].at[pl.ds(r, KCH)], idx_v.at[buf].at[0],
                         semI)
        pltpu.async_copy(ei_hbm.at[1].at[pl.ds(r, KCH)], idx_v.at[buf].at[1],
                         semI)

    def wait_idx(buf):
        for s in range(2):
            pltpu.make_async_copy(ei_hbm.at[0].at[pl.ds(0, KCH)],
                                  idx_v.at[buf].at[s], semI).wait()

    def fire_gathers(buf):
        for j in range(KCH):
            pltpu.async_copy(y_hbm.at[idx_v.at[buf].at[0].at[j]],
                             msg_v.at[buf].at[j], semG)

    def drain_gathers(buf):
        for j in range(KCH):
            pltpu.make_async_copy(y_hbm.at[pl.ds(0, CH)],
                                  msg_v.at[buf].at[j], semG).wait()

    pltpu.sync_copy(ei_hbm.at[0].at[pl.ds(c0, KCH)], idx_v.at[0].at[0])
    pltpu.sync_copy(ei_hbm.at[1].at[pl.ds(c0, KCH)], idx_v.at[0].at[1])
    fire_gathers(0)
    start_idx(1, 1)

    @pl.loop(0, FULLB // 2)
    def _(t):
        for par in (0, 1):
            b = t * 2 + par
            drain_gathers(par)

            @pl.when(b < FULLB - 1)
            def _fire_next():
                wait_idx(1 - par)
                fire_gathers(1 - par)

            for j in range(KCH):
                pltpu.sync_copy(msg_v.at[par].at[j],
                                agg_sh.at[idx_v.at[par].at[1].at[j]],
                                add=True)

            @pl.when(b < FULLB - 2)
            def _prefetch_idx():
                start_idx(b + 2, par)

    for j in range(KCH - 1):
        t = FULLB * KCH + j

        @pl.when(t < nc)
        def _tail():
            r = c0 + t
            pltpu.sync_copy(ei_hbm.at[0].at[pl.ds(r, 1)],
                            idx_v.at[0].at[0].at[pl.ds(j, 1)])
            pltpu.sync_copy(ei_hbm.at[1].at[pl.ds(r, 1)],
                            idx_v.at[0].at[1].at[pl.ds(j, 1)])
            pltpu.async_copy(y_hbm.at[idx_v.at[0].at[0].at[j]],
                             msg_v.at[0].at[j], semG).wait()
            pltpu.sync_copy(msg_v.at[0].at[j],
                            agg_sh.at[idx_v.at[0].at[1].at[j]], add=True)

    plsc.subcore_barrier()
    pltpu.sync_copy(agg_sh.at[pl.ds(sid * RPS, RPS)],
                    out_hbm.at[cid].at[pl.ds(sid * RPS, RPS)])


def _bdot(a, bT):
    return jnp.dot(a.astype(jnp.bfloat16), bT.astype(jnp.bfloat16),
                   preferred_element_type=jnp.float32)


def _tc_layer(parts, h_prev, bd_rel, bd_root, b_tile):
    def body(p_ref, h_ref, wrel_ref, wroot_ref, b_ref, ho_ref):
        agg = p_ref[0] + p_ref[1]
        t = (_bdot(agg, wrel_ref[...]) + b_ref[...]
             + _bdot(h_ref[...], wroot_ref[...]))
        ho_ref[...] = jnp.maximum(t, 0.0)

    return pl.pallas_call(
        body,
        grid=(NPK // BRP,),
        in_specs=[
            pl.BlockSpec((NC, BRP, 128), lambda i: (0, i, 0)),
            pl.BlockSpec((BRP, 128), lambda i: (i, 0)),
            pl.BlockSpec((128, 128), lambda i: (0, 0)),
            pl.BlockSpec((128, 128), lambda i: (0, 0)),
            pl.BlockSpec((1, 128), lambda i: (0, 0)),
        ],
        out_specs=pl.BlockSpec((BRP, 128), lambda i: (i, 0)),
        out_shape=jax.ShapeDtypeStruct((NPK, 128), jnp.float32),
    )(parts, h_prev, bd_rel, bd_root, b_tile)


def _tc_final(parts, h_prev, bd_rel, bd_root, b_tile, bd_m1, bm1_tile,
              wm2_tile, grp, bm2_tile):
    def body(p_ref, h_ref, wrel_ref, wroot_ref, b_ref, wm1_ref, bm1_ref,
             wm2_ref, g_ref, bm2_ref, ho_ref, qo_ref):
        agg = p_ref[0] + p_ref[1]
        t = (_bdot(agg, wrel_ref[...]) + b_ref[...]
             + _bdot(h_ref[...], wroot_ref[...]))
        h = jnp.maximum(t, 0.0)
        ho_ref[...] = h
        m = jnp.maximum(_bdot(h, wm1_ref[...]) + bm1_ref[...], 0.0)
        mb = m.astype(jnp.bfloat16).astype(jnp.float32)
        wb = wm2_ref[...].astype(jnp.bfloat16).astype(jnp.float32)
        prod = mb * wb
        hi = prod.astype(jnp.bfloat16)
        lo = (prod - hi.astype(jnp.float32)).astype(jnp.bfloat16)
        g = g_ref[...].astype(jnp.bfloat16)
        q8 = (jnp.dot(hi, g, preferred_element_type=jnp.float32)
              + jnp.dot(lo, g, preferred_element_type=jnp.float32))
        qo_ref[...] = q8 + bm2_ref[...]

    return pl.pallas_call(
        body,
        grid=(NPK // BRP,),
        in_specs=[
            pl.BlockSpec((NC, BRP, 128), lambda i: (0, i, 0)),
            pl.BlockSpec((BRP, 128), lambda i: (i, 0)),
            pl.BlockSpec((128, 128), lambda i: (0, 0)),
            pl.BlockSpec((128, 128), lambda i: (0, 0)),
            pl.BlockSpec((1, 128), lambda i: (0, 0)),
            pl.BlockSpec((128, 128), lambda i: (0, 0)),
            pl.BlockSpec((1, 128), lambda i: (0, 0)),
            pl.BlockSpec((1, 128), lambda i: (0, 0)),
            pl.BlockSpec((128, PACK), lambda i: (0, 0)),
            pl.BlockSpec((1, PACK), lambda i: (0, 0)),
        ],
        out_specs=[
            pl.BlockSpec((BRP, 128), lambda i: (i, 0)),
            pl.BlockSpec((BRP, PACK), lambda i: (i, 0)),
        ],
        out_shape=[
            jax.ShapeDtypeStruct((NPK, 128), jnp.float32),
            jax.ShapeDtypeStruct((NPK, PACK), jnp.float32),
        ],
    )(parts, h_prev, bd_rel, bd_root, b_tile, bd_m1, bm1_tile, wm2_tile,
      grp, bm2_tile)


def _bd(W):
    WT = W.T
    if WT.shape[0] < F:
        WT = jnp.pad(WT, ((0, F - WT.shape[0]), (0, 0)))
    return jnp.kron(jnp.eye(PACK, dtype=jnp.float32), WT)


def kernel(x, edge_index, W_rel1, W_root1, b1, W_rel2, W_root2, b2,
           W_rel3, W_root3, b3, Wm1, bm1, Wm2, bm2):
    ei = edge_index.reshape(2, NCHUNK, CH)
    zeros_sub = jnp.zeros((RPS, F), jnp.float32)

    def seg(h_packed):
        p = _sc_segment_sum(h_packed.reshape(N_PAD, F), ei, zeros_sub)
        return p.reshape(NC, NPK, 128)

    x16 = jnp.pad(x, ((0, N_PAD - N), (0, F - x.shape[1]))).reshape(NPK, 128)

    def tile(v):
        return jnp.tile(v.reshape(-1), PACK).reshape(1, 128)

    grp = jnp.kron(jnp.eye(PACK, dtype=jnp.float32),
                   jnp.ones((F, 1), jnp.float32))
    bm2_tile = jnp.tile(bm2, PACK).reshape(1, PACK)

    p0 = seg(x16)
    h1 = _tc_layer(p0, x16, _bd(W_rel1), _bd(W_root1), tile(b1))
    p1 = seg(h1)
    h2 = _tc_layer(p1, h1, _bd(W_rel2), _bd(W_root2), tile(b2))
    p2 = seg(h2)
    h3p, qp = _tc_final(p2, h2, _bd(W_rel3), _bd(W_root3), tile(b3),
                        _bd(Wm1), tile(bm1), tile(Wm2), grp, bm2_tile)
    h3 = h3p.reshape(N_PAD, F)[:N]
    q = qp.reshape(N_PAD, 1)[:N]
    return (h3, q)
